# Initial kernel scaffold; baseline (speedup 1.0000x reference)
import jax
import jax.numpy as jnp
from jax.experimental import pallas as pl


def kernel(invoxel_xyz, invoxel_map, src_feat):
    raise NotImplementedError("write your pallas kernel here")


# speedup vs baseline: 11.4507x; 11.4507x over previous
import functools

import jax
import jax.numpy as jnp
from jax import lax
from jax.experimental import pallas as pl
from jax.experimental.pallas import tpu as pltpu
from jax.experimental.pallas import tpu_sc as plsc

N_VOX = 50000
K = 20
D = 64
NUM_POINTS = 200000

C = 40
IDX_PER_CHUNK = C * K
NCHUNK = N_VOX // C
NW = 32
CHUNKS_PER_W = -(-NCHUNK // NW)
GATHER_B = 80
NGATHER = IDX_PER_CHUNK // GATHER_B
LANES = 16

_mesh = plsc.VectorSubcoreMesh(core_axis_name="c", subcore_axis_name="s")


@functools.partial(
    pl.kernel,
    mesh=_mesh,
    compiler_params=pltpu.CompilerParams(
        use_tc_tiling_on_sc=False, needs_layout_passes=False),
    out_type=jax.ShapeDtypeStruct((N_VOX, D), jnp.float32),
    scratch_types=[
        pltpu.VMEM((IDX_PER_CHUNK,), jnp.int32),
        pltpu.VMEM((IDX_PER_CHUNK, D), jnp.float32),
        pltpu.VMEM((C, D), jnp.float32),
        pltpu.SemaphoreType.DMA,
    ],
)
def _pool(map_hbm, feat_hbm, out_hbm, idx_v, rows_v, out_v, sem):
    wid = lax.axis_index("s") * 2 + lax.axis_index("c")

    def do_chunk(ci, carry):
        c = wid + ci * NW

        @pl.when(c < NCHUNK)
        def _():
            base = c * IDX_PER_CHUNK
            pltpu.sync_copy(map_hbm.at[pl.ds(base, IDX_PER_CHUNK)], idx_v)

            def fix(i, carry2):
                sl = pl.ds(i * LANES, LANES)
                vals = idx_v[sl]
                p = i * LANES + lax.iota(jnp.int32, LANES)
                first_pos = lax.div(p, K) * K
                first = plsc.load_gather(idx_v, [first_pos])
                idx_v[sl] = jnp.where(vals == 0, first, vals)
                return carry2

            lax.fori_loop(0, IDX_PER_CHUNK // LANES, fix, 0)

            copies = []
            for j in range(NGATHER):
                copies.append(pltpu.async_copy(
                    feat_hbm.at[idx_v.at[pl.ds(j * GATHER_B, GATHER_B)]],
                    rows_v.at[pl.ds(j * GATHER_B, GATHER_B), :],
                    sem,
                ))
            for cp in copies:
                cp.wait()

            def pool_one(v, carry2):
                rbase = v * K
                for d in range(D // LANES):
                    sl = pl.ds(d * LANES, LANES)
                    acc = rows_v[rbase, sl]
                    for k in range(1, K):
                        acc = acc + rows_v[rbase + k, sl]
                    out_v[v, sl] = acc * (1.0 / K)
                return carry2

            lax.fori_loop(0, C, pool_one, 0)

            pltpu.sync_copy(out_v, out_hbm.at[pl.ds(c * C, C), :])

        return carry

    lax.fori_loop(0, CHUNKS_PER_W, do_chunk, 0)


def kernel(invoxel_xyz, invoxel_map, src_feat):
    del invoxel_xyz
    map_flat = invoxel_map.reshape(-1)
    return _pool(map_flat, src_feat)


# speedup vs baseline: 14.1948x; 1.2397x over previous
import functools

import jax
import jax.numpy as jnp
from jax import lax
from jax.experimental import pallas as pl
from jax.experimental.pallas import tpu as pltpu
from jax.experimental.pallas import tpu_sc as plsc

N_VOX = 50000
K = 20
D = 64
NUM_POINTS = 200000

C = 40
IDX_PER_CHUNK = C * K
NCHUNK = N_VOX // C
NW = 32
CHUNKS_PER_W = -(-NCHUNK // NW)
GATHER_B = 80
NGATHER = IDX_PER_CHUNK // GATHER_B
LANES = 16

_mesh = plsc.VectorSubcoreMesh(core_axis_name="c", subcore_axis_name="s")


@functools.partial(
    pl.kernel,
    mesh=_mesh,
    compiler_params=pltpu.CompilerParams(
        use_tc_tiling_on_sc=False, needs_layout_passes=False),
    out_type=jax.ShapeDtypeStruct((N_VOX, D), jnp.float32),
    scratch_types=[
        pltpu.VMEM((IDX_PER_CHUNK,), jnp.int32),
        pltpu.VMEM((IDX_PER_CHUNK,), jnp.int32),
        pltpu.VMEM((IDX_PER_CHUNK, D), jnp.float32),
        pltpu.VMEM((IDX_PER_CHUNK, D), jnp.float32),
        pltpu.VMEM((C, D), jnp.float32),
        pltpu.VMEM((C, D), jnp.float32),
        pltpu.SemaphoreType.DMA,
        pltpu.SemaphoreType.DMA,
    ],
)
def _pool(map_hbm, feat_hbm, out_hbm, idx0, idx1, rows0, rows1, out0, out1,
          sem0, sem1):
    wid = lax.axis_index("s") * 2 + lax.axis_index("c")
    bufs = ((idx0, rows0, out0, sem0), (idx1, rows1, out1, sem1))

    def fire(c, idx_v, rows_v, sem):
        base = c * IDX_PER_CHUNK
        pltpu.sync_copy(map_hbm.at[pl.ds(base, IDX_PER_CHUNK)], idx_v)

        def fix(i, carry):
            sl = pl.ds(i * LANES, LANES)
            vals = idx_v[sl]
            p = i * LANES + lax.iota(jnp.int32, LANES)
            first_pos = lax.div(p, K) * K
            first = plsc.load_gather(idx_v, [first_pos])
            idx_v[sl] = jnp.where(vals == 0, first, vals)
            return carry

        lax.fori_loop(0, IDX_PER_CHUNK // LANES, fix, 0)

        for j in range(NGATHER):
            pltpu.async_copy(
                feat_hbm.at[idx_v.at[pl.ds(j * GATHER_B, GATHER_B)]],
                rows_v.at[pl.ds(j * GATHER_B, GATHER_B), :],
                sem,
            )

    def drain(idx_v, rows_v, sem):
        for j in range(NGATHER):
            pltpu.make_async_copy(
                feat_hbm.at[idx_v.at[pl.ds(j * GATHER_B, GATHER_B)]],
                rows_v.at[pl.ds(j * GATHER_B, GATHER_B), :],
                sem,
            ).wait()

    def reduce_store(c, rows_v, out_v):
        def pool_one(v, carry):
            rbase = v * K
            for d in range(D // LANES):
                sl = pl.ds(d * LANES, LANES)
                acc = rows_v[rbase, sl]
                for k in range(1, K):
                    acc = acc + rows_v[rbase + k, sl]
                out_v[v, sl] = acc * (1.0 / K)
            return carry

        lax.fori_loop(0, C, pool_one, 0)
        pltpu.sync_copy(out_v, out_hbm.at[pl.ds(c * C, C), :])

    fire(wid, *bufs[0][:3][:2], bufs[0][3])

    def pair(ii, carry):
        for b in range(2):
            i_cur = ii * 2 + b
            c_cur = wid + i_cur * NW
            c_next = c_cur + NW
            nb = bufs[1 - b]
            cb = bufs[b]

            @pl.when(c_next < NCHUNK)
            def _():
                fire(c_next, nb[0], nb[1], nb[3])

            @pl.when(c_cur < NCHUNK)
            def _():
                drain(cb[0], cb[1], cb[3])
                reduce_store(c_cur, cb[1], cb[2])

        return carry

    lax.fori_loop(0, CHUNKS_PER_W // 2, pair, 0)


def kernel(invoxel_xyz, invoxel_map, src_feat):
    del invoxel_xyz
    map_flat = invoxel_map.reshape(-1)
    return _pool(map_flat, src_feat)


# speedup vs baseline: 15.3068x; 1.0783x over previous
import functools

import jax
import jax.numpy as jnp
from jax import lax
from jax.experimental import pallas as pl
from jax.experimental.pallas import tpu as pltpu
from jax.experimental.pallas import tpu_sc as plsc

N_VOX = 50000
K = 20
D = 64
NUM_POINTS = 200000

C = 40
IDX_PER_CHUNK = C * K
NCHUNK = N_VOX // C
NW = 32
CHUNKS_PER_W = -(-NCHUNK // NW)
GATHER_B = 80
NGATHER = IDX_PER_CHUNK // GATHER_B
LANES = 16

_mesh = plsc.VectorSubcoreMesh(core_axis_name="c", subcore_axis_name="s")


@functools.partial(
    pl.kernel,
    mesh=_mesh,
    compiler_params=pltpu.CompilerParams(
        use_tc_tiling_on_sc=False, needs_layout_passes=False),
    out_type=jax.ShapeDtypeStruct((N_VOX, D), jnp.float32),
    scratch_types=[
        pltpu.VMEM((IDX_PER_CHUNK,), jnp.int32),
        pltpu.VMEM((IDX_PER_CHUNK,), jnp.int32),
        pltpu.VMEM((IDX_PER_CHUNK,), jnp.int32),
        pltpu.VMEM((IDX_PER_CHUNK,), jnp.int32),
        pltpu.VMEM((IDX_PER_CHUNK, D), jnp.float32),
        pltpu.VMEM((IDX_PER_CHUNK, D), jnp.float32),
        pltpu.VMEM((C, D), jnp.float32),
        pltpu.VMEM((C, D), jnp.float32),
        pltpu.SemaphoreType.DMA,
        pltpu.SemaphoreType.DMA,
        pltpu.SemaphoreType.DMA,
        pltpu.SemaphoreType.DMA,
        pltpu.SemaphoreType.DMA,
        pltpu.SemaphoreType.DMA,
    ],
)
def _pool(map_hbm, feat_hbm, out_hbm,
          raw0, raw1, fix0, fix1, rows0, rows1, out0, out1,
          msem0, msem1, gsem0, gsem1, osem0, osem1):
    wid = lax.axis_index("s") * 2 + lax.axis_index("c")
    raw = (raw0, raw1)
    fixv = (fix0, fix1)
    rows = (rows0, rows1)
    outv = (out0, out1)
    msem = (msem0, msem1)
    gsem = (gsem0, gsem1)
    osem = (osem0, osem1)

    def start_map(c, b):
        pltpu.async_copy(
            map_hbm.at[pl.ds(c * IDX_PER_CHUNK, IDX_PER_CHUNK)], raw[b],
            msem[b])

    def front(c, b):
        pltpu.make_async_copy(
            map_hbm.at[pl.ds(c * IDX_PER_CHUNK, IDX_PER_CHUNK)], raw[b],
            msem[b]).wait()

        for i in range(IDX_PER_CHUNK // LANES):
            sl = pl.ds(i * LANES, LANES)
            vals = raw[b][sl]
            p = i * LANES + lax.iota(jnp.int32, LANES)
            first = plsc.load_gather(raw[b], [lax.div(p, K) * K])
            fixv[b][sl] = jnp.where(vals == 0, first, vals)

        for j in range(NGATHER):
            pltpu.async_copy(
                feat_hbm.at[fixv[b].at[pl.ds(j * GATHER_B, GATHER_B)]],
                rows[b].at[pl.ds(j * GATHER_B, GATHER_B), :],
                gsem[b],
            )

        @pl.when(c + 2 * NW < NCHUNK)
        def _():
            start_map(c + 2 * NW, b)

    def back(c, b):
        for j in range(NGATHER):
            pltpu.make_async_copy(
                feat_hbm.at[fixv[b].at[pl.ds(j * GATHER_B, GATHER_B)]],
                rows[b].at[pl.ds(j * GATHER_B, GATHER_B), :],
                gsem[b],
            ).wait()

        @pl.when(c >= 2 * NW)
        def _():
            pltpu.make_async_copy(
                outv[b], out_hbm.at[pl.ds(c * C, C), :], osem[b]).wait()

        def pool_one(v, carry):
            rbase = v * K
            for d in range(D // LANES):
                sl = pl.ds(d * LANES, LANES)
                acc = rows[b][rbase, sl]
                for k in range(1, K):
                    acc = acc + rows[b][rbase + k, sl]
                outv[b][v, sl] = acc * (1.0 / K)
            return carry

        lax.fori_loop(0, C, pool_one, 0)
        pltpu.async_copy(outv[b], out_hbm.at[pl.ds(c * C, C), :], osem[b])

    start_map(wid, 0)
    start_map(wid + NW, 1)
    front(wid, 0)

    def pair(ii, carry):
        for b in range(2):
            c_cur = wid + (ii * 2 + b) * NW
            c_next = c_cur + NW

            @pl.when(c_next < NCHUNK)
            def _():
                front(c_next, 1 - b)

            @pl.when(c_cur < NCHUNK)
            def _():
                back(c_cur, b)

        return carry

    lax.fori_loop(0, CHUNKS_PER_W // 2, pair, 0)

    for b in range(2):
        pltpu.make_async_copy(
            outv[b], out_hbm.at[pl.ds(0, C), :], osem[b]).wait()


def kernel(invoxel_xyz, invoxel_map, src_feat):
    del invoxel_xyz
    map_flat = invoxel_map.reshape(-1)
    return _pool(map_flat, src_feat)


# speedup vs baseline: 17.8009x; 1.1629x over previous
import functools

import jax
import jax.numpy as jnp
from jax import lax
from jax.experimental import pallas as pl
from jax.experimental.pallas import tpu as pltpu
from jax.experimental.pallas import tpu_sc as plsc

N_VOX = 50000
K = 20
D = 64
NUM_POINTS = 200000

C = 40
IDX_PER_CHUNK = C * K
NCHUNK = N_VOX // C
NW = 32
NSUB = 16
CHUNKS_PER_W = -(-NCHUNK // NW)
GATHER_B = 80
NGATHER = IDX_PER_CHUNK // GATHER_B
NSCAT = NGATHER // 2
NVALU = NGATHER - NSCAT
C_VALU = NVALU * GATHER_B // K
C_SCAT = C - C_VALU
LANES = 16

_mesh = plsc.VectorSubcoreMesh(core_axis_name="c", subcore_axis_name="s")


@functools.partial(
    pl.kernel,
    mesh=_mesh,
    compiler_params=pltpu.CompilerParams(
        use_tc_tiling_on_sc=False, needs_layout_passes=False),
    out_type=jax.ShapeDtypeStruct((N_VOX, D), jnp.float32),
    scratch_types=[
        pltpu.VMEM((IDX_PER_CHUNK,), jnp.int32),
        pltpu.VMEM((IDX_PER_CHUNK,), jnp.int32),
        pltpu.VMEM((IDX_PER_CHUNK,), jnp.int32),
        pltpu.VMEM((IDX_PER_CHUNK,), jnp.int32),
        pltpu.VMEM((IDX_PER_CHUNK, D), jnp.float32),
        pltpu.VMEM((IDX_PER_CHUNK, D), jnp.float32),
        pltpu.VMEM((C, D), jnp.float32),
        pltpu.VMEM((C, D), jnp.float32),
        pltpu.VMEM((C_SCAT, D), jnp.float32),
        pltpu.VMEM((NSCAT, GATHER_B), jnp.int32),
        pltpu.VMEM_SHARED((NSUB, 2, C_SCAT, D), jnp.float32),
        pltpu.SemaphoreType.DMA,
        pltpu.SemaphoreType.DMA,
        pltpu.SemaphoreType.DMA,
        pltpu.SemaphoreType.DMA,
        pltpu.SemaphoreType.DMA,
        pltpu.SemaphoreType.DMA,
        pltpu.SemaphoreType.DMA,
        pltpu.SemaphoreType.DMA,
        pltpu.SemaphoreType.DMA,
        pltpu.SemaphoreType.DMA,
        pltpu.SemaphoreType.DMA,
        pltpu.SemaphoreType.DMA,
    ],
)
def _pool(map_hbm, feat_hbm, out_hbm,
          raw0, raw1, fix0, fix1, rows0, rows1, out0, out1, zeros_v,
          dst_idx, acc_sh,
          msem0, msem1, gasem0, gasem1, gbsem0, gbsem1, ssem0, ssem1,
          osem0, osem1, zsem0, zsem1):
    sid = lax.axis_index("s")
    wid = sid * 2 + lax.axis_index("c")
    raw = (raw0, raw1)
    fixv = (fix0, fix1)
    rows = (rows0, rows1)
    outv = (out0, out1)
    accv = (acc_sh.at[sid, 0], acc_sh.at[sid, 1])
    msem = (msem0, msem1)
    gasem = (gasem0, gasem1)
    gbsem = (gbsem0, gbsem1)
    ssem = (ssem0, ssem1)
    osem = (osem0, osem1)
    zsem = (zsem0, zsem1)

    zvec = jnp.zeros((LANES,), jnp.float32)
    for v in range(C_SCAT):
        for d in range(D // LANES):
            zeros_v[v, pl.ds(d * LANES, LANES)] = zvec

    for jj in range(NSCAT):
        for t in range(GATHER_B // LANES):
            base = (NVALU + jj) * GATHER_B + t * LANES
            vals = lax.div(base + lax.iota(jnp.int32, LANES), K) - C_VALU
            dst_idx[jj, pl.ds(t * LANES, LANES)] = vals

    def start_map(c, b):
        pltpu.async_copy(
            map_hbm.at[pl.ds(c * IDX_PER_CHUNK, IDX_PER_CHUNK)], raw[b],
            msem[b])

    def gather_batch(b, j, sem):
        return (feat_hbm.at[fixv[b].at[pl.ds(j * GATHER_B, GATHER_B)]],
                rows[b].at[pl.ds(j * GATHER_B, GATHER_B), :], sem)

    def front(c, b):
        pltpu.make_async_copy(
            map_hbm.at[pl.ds(c * IDX_PER_CHUNK, IDX_PER_CHUNK)], raw[b],
            msem[b]).wait()

        for i in range(IDX_PER_CHUNK // LANES):
            sl = pl.ds(i * LANES, LANES)
            vals = raw[b][sl]
            p = i * LANES + lax.iota(jnp.int32, LANES)
            first = plsc.load_gather(raw[b], [lax.div(p, K) * K])
            fixv[b][sl] = jnp.where(vals == 0, first, vals)

        for j in range(NVALU, NGATHER):
            pltpu.async_copy(*gather_batch(b, j, gasem[b]))
        for j in range(NVALU):
            pltpu.async_copy(*gather_batch(b, j, gbsem[b]))

        @pl.when(c + 2 * NW < NCHUNK)
        def _():
            start_map(c + 2 * NW, b)

    def back(c, b):
        for j in range(NVALU, NGATHER):
            pltpu.make_async_copy(*gather_batch(b, j, gasem[b])).wait()
        pltpu.make_async_copy(zeros_v, accv[b], zsem[b]).wait()

        for jj in range(NSCAT):
            pltpu.async_copy(
                rows[b].at[pl.ds((NVALU + jj) * GATHER_B, GATHER_B), :],
                accv[b].at[dst_idx.at[jj]],
                ssem[b],
                add=True,
            )

        for j in range(NVALU):
            pltpu.make_async_copy(*gather_batch(b, j, gbsem[b])).wait()

        @pl.when(c >= 2 * NW)
        def _():
            pltpu.make_async_copy(
                outv[b], out_hbm.at[pl.ds(c * C, C), :], osem[b]).wait()

        def pool_one(v, carry):
            rbase = v * K
            for d in range(D // LANES):
                sl = pl.ds(d * LANES, LANES)
                acc = rows[b][rbase, sl]
                for k in range(1, K):
                    acc = acc + rows[b][rbase + k, sl]
                outv[b][v, sl] = acc * (1.0 / K)
            return carry

        lax.fori_loop(0, C_VALU, pool_one, 0)

        for jj in range(NSCAT):
            pltpu.make_async_copy(
                rows[b].at[pl.ds((NVALU + jj) * GATHER_B, GATHER_B), :],
                accv[b].at[dst_idx.at[jj]],
                ssem[b],
            ).wait()

        pltpu.sync_copy(accv[b], outv[b].at[pl.ds(C_VALU, C_SCAT), :])
        pltpu.async_copy(zeros_v, accv[b], zsem[b])

        for v in range(C_SCAT):
            for d in range(D // LANES):
                sl = pl.ds(d * LANES, LANES)
                outv[b][C_VALU + v, sl] = outv[b][C_VALU + v, sl] * (1.0 / K)

        pltpu.async_copy(outv[b], out_hbm.at[pl.ds(c * C, C), :], osem[b])

    start_map(wid, 0)
    start_map(wid + NW, 1)
    pltpu.async_copy(zeros_v, accv[0], zsem[0])
    pltpu.async_copy(zeros_v, accv[1], zsem[1])
    front(wid, 0)

    def pair(ii, carry):
        for b in range(2):
            c_cur = wid + (ii * 2 + b) * NW
            c_next = c_cur + NW

            @pl.when(c_next < NCHUNK)
            def _():
                front(c_next, 1 - b)

            @pl.when(c_cur < NCHUNK)
            def _():
                back(c_cur, b)

        return carry

    lax.fori_loop(0, CHUNKS_PER_W // 2, pair, 0)

    for b in range(2):
        pltpu.make_async_copy(
            outv[b], out_hbm.at[pl.ds(0, C), :], osem[b]).wait()
        pltpu.make_async_copy(zeros_v, accv[b], zsem[b]).wait()


def kernel(invoxel_xyz, invoxel_map, src_feat):
    del invoxel_xyz
    map_flat = invoxel_map.reshape(-1)
    return _pool(map_flat, src_feat)
